# baseline (speedup 1.0000x reference)
import functools

import jax
import jax.numpy as jnp
from jax import lax
from jax.experimental import pallas as pl
from jax.experimental.pallas import tpu as pltpu
from jax.experimental.pallas import tpu_sc as plsc


_LANES = 16
_LPAD = 64
_VPAD = 1024
_SPAD = 56


def _lse_body(w_ref, lse_ref):
    w = w_ref[...]
    m = jnp.max(w, axis=1)
    lse_ref[...] = m + jnp.log(jnp.sum(jnp.exp(w - m[:, None]), axis=1))


def _row_lse(W):
    V = W.shape[0]
    return pl.pallas_call(
        _lse_body,
        out_shape=jax.ShapeDtypeStruct((V,), jnp.float32),
    )(W)


def _sc_row_gather(idxg, Wp, *, B, L, V, nw):
    rows = B // nw
    mesh = plsc.VectorSubcoreMesh(core_axis_name="c", subcore_axis_name="s")
    info = plsc.get_sparse_core_info()
    nc = info.num_cores

    @functools.partial(
        pl.kernel,
        mesh=mesh,
        compiler_params=pltpu.CompilerParams(use_tc_tiling_on_sc=True),
        out_type=jax.ShapeDtypeStruct((B, _SPAD, _VPAD), jnp.float32),
        scratch_types=[
            pltpu.VMEM((rows // 2, 2 * _LPAD), jnp.int32),
            pltpu.VMEM((_SPAD, _VPAD), jnp.float32),
            pltpu.VMEM((_SPAD, _VPAD), jnp.float32),
            pltpu.SemaphoreType.DMA,
            pltpu.SemaphoreType.DMA,
            pltpu.SemaphoreType.DMA,
            pltpu.SemaphoreType.DMA,
        ],
    )
    def k(idx_hbm, w_hbm, out_hbm, idx_v, buf0, buf1,
          gsem0, gsem1, ssem0, ssem1):
        wid = lax.axis_index("s") * nc + lax.axis_index("c")
        bufs = (buf0, buf1)
        gsems = (gsem0, gsem1)
        ssems = (ssem0, ssem1)
        pltpu.sync_copy(idx_hbm.at[pl.ds(wid * (rows // 2), rows // 2)], idx_v)

        def gather(i, par, buf, sem):
            ilist = idx_v.at[i, pl.ds(par * _LPAD, _SPAD)]
            return pltpu.make_async_copy(w_hbm.at[ilist], buf, sem)

        def store(i, par, buf, sem):
            b = wid * rows + 2 * i + par
            return pltpu.make_async_copy(buf, out_hbm.at[b], sem)

        gather(0, 0, buf0, gsem0).start()
        gather(0, 1, buf1, gsem1).start()

        def step(i, carry):
            for par in range(2):
                buf, gsem, ssem = bufs[par], gsems[par], ssems[par]
                gather(i, par, buf, gsem).wait()
                store(i, par, buf, ssem).start()

                @pl.when(2 * i + par + 2 < rows)
                def _refill():
                    store(i, par, buf, ssem).wait()
                    gather(i + 1, par, buf, gsem).start()

            return carry

        lax.fori_loop(0, rows // 2, step, 0)
        store(rows // 2 - 1, 0, buf0, ssem0).wait()
        store(rows // 2 - 1, 1, buf1, ssem1).wait()

    return k(idxg, Wp)


def _sc_loss(idx2, tgt2, Wflat, lse, *, B, L, V, nw):
    per_w = (B // nw) * _LPAD
    nseg = per_w // 128
    mesh = plsc.VectorSubcoreMesh(core_axis_name="c", subcore_axis_name="s")
    info = plsc.get_sparse_core_info()
    nc = info.num_cores

    @functools.partial(
        pl.kernel,
        mesh=mesh,
        compiler_params=pltpu.CompilerParams(
            use_tc_tiling_on_sc=False, needs_layout_passes=False),
        out_type=[
            jax.ShapeDtypeStruct((nw, _LANES), jnp.float32),
            jax.ShapeDtypeStruct((nw, _LANES), jnp.float32),
        ],
        scratch_types=[
            pltpu.VMEM((per_w,), jnp.int32),
            pltpu.VMEM((per_w,), jnp.int32),
            pltpu.VMEM((per_w,), jnp.int32),
            pltpu.VMEM((per_w,), jnp.float32),
            pltpu.VMEM((V,), jnp.float32),
            pltpu.VMEM((_LANES,), jnp.float32),
            pltpu.VMEM((_LANES,), jnp.float32),
            pltpu.SemaphoreType.DMA,
        ],
    )
    def k(idx_hbm, tgt_hbm, wf_hbm, lse_hbm, acc_hbm, cnt_hbm,
          idx_v, tgt_v, fidx_v, wit_v, lse_v, acc_v, cnt_v, esem):
        wid = lax.axis_index("s") * nc + lax.axis_index("c")
        pltpu.sync_copy(idx_hbm.at[wid], idx_v)
        pltpu.sync_copy(tgt_hbm.at[wid], tgt_v)
        pltpu.sync_copy(lse_hbm, lse_v)
        acc_v[...] = jnp.zeros((_LANES,), jnp.float32)
        cnt_v[...] = jnp.zeros((_LANES,), jnp.float32)

        def fbody(r, carry):
            for j in range(128 // _LANES):
                sl = pl.ds(r * 128 + j * _LANES, _LANES)
                i16 = jnp.clip(idx_v[sl], 0, V - 1)
                t16 = tgt_v[sl]
                tsafe = jnp.where(t16 != -1, t16, 0)
                fidx_v[sl] = i16 * V + tsafe
            return carry

        lax.fori_loop(0, nseg, fbody, 0)

        def gbody(r, carry):
            sl = pl.ds(r * 128, 128)
            pltpu.make_async_copy(
                wf_hbm.at[fidx_v.at[sl]], wit_v.at[sl], esem).start()
            return carry

        lax.fori_loop(0, nseg, gbody, 0)

        def wbody(r, carry):
            sl = pl.ds(r * 128, 128)
            pltpu.make_async_copy(
                wf_hbm.at[fidx_v.at[sl]], wit_v.at[sl], esem).wait()
            return carry

        lax.fori_loop(0, nseg, wbody, 0)

        def rbody(r, carry):
            for j in range(128 // _LANES):
                sl = pl.ds(r * 128 + j * _LANES, _LANES)
                i16 = jnp.clip(idx_v[sl], 0, V - 1)
                t16 = tgt_v[sl]
                m = t16 != -1
                wit = wit_v[sl]
                ls16 = plsc.load_gather(lse_v, [i16])
                acc_v[...] = acc_v[...] + jnp.where(m, wit - ls16, 0.0)
                cnt_v[...] = cnt_v[...] + jnp.where(m, 1.0, 0.0)
            return carry

        lax.fori_loop(0, nseg, rbody, 0)
        pltpu.sync_copy(acc_v, acc_hbm.at[wid])
        pltpu.sync_copy(cnt_v, cnt_hbm.at[wid])

    return k(idx2, tgt2, Wflat, lse)


def _fin_body(acc_ref, cnt_ref, out_ref):
    s = jnp.sum(acc_ref[...])
    c = jnp.sum(cnt_ref[...])
    out_ref[...] = jnp.full((1, 1), -(s / jnp.maximum(c, 1.0)), jnp.float32)


def _finalize(acc, cnt):
    return pl.pallas_call(
        _fin_body,
        out_shape=jax.ShapeDtypeStruct((1, 1), jnp.float32),
    )(acc, cnt)


def kernel(idx, targets, W):
    B, L = idx.shape
    V = W.shape[0]
    info = plsc.get_sparse_core_info()
    nw = info.num_cores * info.num_subcores
    assert B % (2 * nw) == 0 and L <= _LPAD and V <= _VPAD

    rows = B // nw
    idx_p = jnp.pad(idx.astype(jnp.int32), ((0, 0), (0, _LPAD - L)))
    idxg = idx_p.reshape(nw * rows * _LPAD // 128, 128)
    idx2 = idx_p.reshape(nw, rows * _LPAD)
    tgt2 = jnp.pad(targets.astype(jnp.int32), ((0, 0), (0, _LPAD - L)),
                   constant_values=-1).reshape(nw, rows * _LPAD)
    Wp = jnp.pad(W, ((0, 0), (0, _VPAD - V)))
    Wflat = W.reshape(V * V)

    logits = _sc_row_gather(idxg, Wp, B=B, L=L, V=V, nw=nw)[:, :L, :V]
    lse = _row_lse(W)
    acc, cnt = _sc_loss(idx2, tgt2, Wflat, lse, B=B, L=L, V=V, nw=nw)
    loss = _finalize(acc, cnt)[0, 0]
    return logits, loss


# speedup vs baseline: 1.5132x; 1.5132x over previous
import functools

import jax
import jax.numpy as jnp
from jax import lax
from jax.experimental import pallas as pl
from jax.experimental.pallas import tpu as pltpu
from jax.experimental.pallas import tpu_sc as plsc


def _lse_body(w_ref, lse_ref):
    w = w_ref[...]
    m = jnp.max(w, axis=1)
    lse_ref[...] = m + jnp.log(jnp.sum(jnp.exp(w - m[:, None]), axis=1))


def _row_lse(W):
    V = W.shape[0]
    return pl.pallas_call(
        _lse_body,
        out_shape=jax.ShapeDtypeStruct((V,), jnp.float32),
    )(W)


_LANES = 16


_LPAD = 64
_SPAD = 56
_VPAD = 1024


def _sc_gather(idx2, tgt2, Wp, lse, *, B, L, nw):
    V = Wp.shape[0]
    rows = B // nw
    mesh = plsc.VectorSubcoreMesh(core_axis_name="c", subcore_axis_name="s")
    info = plsc.get_sparse_core_info()
    nc = info.num_cores
    groups = (L + _LANES - 1) // _LANES

    @functools.partial(
        pl.kernel,
        mesh=mesh,
        compiler_params=pltpu.CompilerParams(
            use_tc_tiling_on_sc=False, needs_layout_passes=False),
        out_type=[
            jax.ShapeDtypeStruct((B, _SPAD, _VPAD), jnp.float32),
            jax.ShapeDtypeStruct((nw, _LANES), jnp.float32),
            jax.ShapeDtypeStruct((nw, _LANES), jnp.float32),
        ],
        scratch_types=[
            pltpu.VMEM((rows * _LPAD,), jnp.int32),
            pltpu.VMEM((rows * _LPAD,), jnp.int32),
            pltpu.VMEM((V,), jnp.float32),
            pltpu.VMEM((L, _VPAD), jnp.float32),
            pltpu.VMEM((L, _VPAD), jnp.float32),
            pltpu.VMEM((_LANES,), jnp.float32),
            pltpu.VMEM((_LANES,), jnp.float32),
            pltpu.SemaphoreType.DMA,
            pltpu.SemaphoreType.DMA,
            pltpu.SemaphoreType.DMA,
            pltpu.SemaphoreType.DMA,
        ],
    )
    def k(idx_hbm, tgt_hbm, w_hbm, lse_hbm, out_hbm, acc_hbm, cnt_hbm,
          idx_v, tgt_v, lse_v, buf0, buf1, acc_v, cnt_v,
          gsem0, gsem1, ssem0, ssem1):
        wid = lax.axis_index("s") * nc + lax.axis_index("c")
        bufs = (buf0, buf1)
        gsems = (gsem0, gsem1)
        ssems = (ssem0, ssem1)
        pltpu.sync_copy(idx_hbm.at[wid], idx_v)
        pltpu.sync_copy(tgt_hbm.at[wid], tgt_v)
        pltpu.sync_copy(lse_hbm, lse_v)
        acc_v[...] = jnp.zeros((_LANES,), jnp.float32)
        cnt_v[...] = jnp.zeros((_LANES,), jnp.float32)

        def gather(g, buf, sem):
            ilist = idx_v.at[pl.ds(g * _LPAD, L)]
            return pltpu.make_async_copy(w_hbm.at[ilist], buf, sem)

        def store(g, buf, sem):
            return pltpu.make_async_copy(
                buf, out_hbm.at[wid * rows + g, pl.ds(0, L)], sem)

        gather(0, buf0, gsem0).start()
        gather(1, buf1, gsem1).start()

        def step(i, carry):
            for par in range(2):
                g = 2 * i + par
                buf, gsem, ssem = bufs[par], gsems[par], ssems[par]
                gather(g, buf, gsem).wait()
                for j in range(groups):
                    sl = pl.ds(g * _LPAD + j * _LANES, _LANES)
                    lanes = jnp.arange(_LANES, dtype=jnp.int32) + j * _LANES
                    i16 = jnp.clip(idx_v[sl], 0, V - 1)
                    t16 = tgt_v[sl]
                    m = t16 != -1
                    tsafe = jnp.where(m, t16, 0)
                    row16 = jnp.minimum(lanes, L - 1)
                    wit = plsc.load_gather(buf, [row16, tsafe])
                    ls16 = plsc.load_gather(lse_v, [i16])
                    acc_v[...] = acc_v[...] + jnp.where(m, wit - ls16, 0.0)
                    cnt_v[...] = cnt_v[...] + jnp.where(m, 1.0, 0.0)
                store(g, buf, ssem).start()

                @pl.when(g + 2 < rows)
                def _refill():
                    store(g, buf, ssem).wait()
                    gather(g + 2, buf, gsem).start()

            return carry

        lax.fori_loop(0, rows // 2, step, 0)
        store(rows - 2, buf0, ssem0).wait()
        store(rows - 1, buf1, ssem1).wait()
        pltpu.sync_copy(acc_v, acc_hbm.at[wid])
        pltpu.sync_copy(cnt_v, cnt_hbm.at[wid])

    return k(idx2, tgt2, Wp, lse)


def _fin_body(acc_ref, cnt_ref, out_ref):
    s = jnp.sum(acc_ref[...])
    c = jnp.sum(cnt_ref[...])
    out_ref[...] = jnp.full((1, 1), -(s / jnp.maximum(c, 1.0)), jnp.float32)


def _finalize(acc, cnt):
    return pl.pallas_call(
        _fin_body,
        out_shape=jax.ShapeDtypeStruct((1, 1), jnp.float32),
    )(acc, cnt)


def kernel(idx, targets, W):
    B, L = idx.shape
    V = W.shape[0]
    info = plsc.get_sparse_core_info()
    nw = info.num_cores * info.num_subcores
    assert B % (2 * nw) == 0

    rows = B // nw
    idx2 = jnp.pad(idx.astype(jnp.int32), ((0, 0), (0, _LPAD - L))
                   ).reshape(nw, rows * _LPAD)
    tgt2 = jnp.pad(targets.astype(jnp.int32), ((0, 0), (0, _LPAD - L)),
                   constant_values=-1).reshape(nw, rows * _LPAD)
    Wp = jnp.pad(W, ((0, 0), (0, _VPAD - V)))
    lse = _row_lse(W)
    raw, acc, cnt = _sc_gather(idx2, tgt2, Wp, lse, B=B, L=L, nw=nw)
    logits = raw[:, :L, :V]
    loss = _finalize(acc, cnt)[0, 0]
    return logits, loss


# speedup vs baseline: 2.5049x; 1.6553x over previous
import functools

import jax
import jax.numpy as jnp
from jax import lax
from jax.experimental import pallas as pl
from jax.experimental.pallas import tpu as pltpu
from jax.experimental.pallas import tpu_sc as plsc


def _lse_body(w_ref, lse_ref):
    w = w_ref[...]
    m = jnp.max(w, axis=1)
    lse_ref[...] = m + jnp.log(jnp.sum(jnp.exp(w - m[:, None]), axis=1))


def _row_lse(W):
    V = W.shape[0]
    return pl.pallas_call(
        _lse_body,
        out_shape=jax.ShapeDtypeStruct((V,), jnp.float32),
    )(W)


_LANES = 16


_LPAD = 64
_SPAD = 56
_VPAD = 1024


def _sc_gather(idx2, tgt2, Wp, lse, *, B, L, nw):
    V = Wp.shape[0]
    rows = B // nw
    mesh = plsc.VectorSubcoreMesh(core_axis_name="c", subcore_axis_name="s")
    info = plsc.get_sparse_core_info()
    nc = info.num_cores
    groups = (L + _LANES - 1) // _LANES

    @functools.partial(
        pl.kernel,
        mesh=mesh,
        compiler_params=pltpu.CompilerParams(
            use_tc_tiling_on_sc=True, needs_layout_passes=False),
        out_type=[
            jax.ShapeDtypeStruct((B, _SPAD, _VPAD), jnp.float32),
            jax.ShapeDtypeStruct((nw * 1024,), jnp.float32),
            jax.ShapeDtypeStruct((nw * 1024,), jnp.float32),
        ],
        scratch_types=[
            pltpu.VMEM((rows * _LPAD,), jnp.int32),
            pltpu.VMEM((rows * _LPAD,), jnp.int32),
            pltpu.VMEM((_VPAD,), jnp.float32),
            pltpu.VMEM((_SPAD, 8, 128), jnp.float32),
            pltpu.VMEM((_SPAD, 8, 128), jnp.float32),
            pltpu.VMEM((1024,), jnp.float32),
            pltpu.VMEM((1024,), jnp.float32),
            pltpu.SemaphoreType.DMA,
            pltpu.SemaphoreType.DMA,
            pltpu.SemaphoreType.DMA,
            pltpu.SemaphoreType.DMA,
        ],
    )
    def k(idx_hbm, tgt_hbm, w_hbm, lse_hbm, out_hbm, acc_hbm, cnt_hbm,
          idx_v, tgt_v, lse_v, buf0, buf1, acc_s, cnt_s,
          gsem0, gsem1, ssem0, ssem1):
        wid = lax.axis_index("s") * nc + lax.axis_index("c")
        nstg = rows * _LPAD
        bufs = (buf0, buf1)
        gsems = (gsem0, gsem1)
        ssems = (ssem0, ssem1)
        pltpu.sync_copy(idx_hbm.at[pl.ds(wid * nstg, nstg)], idx_v)
        pltpu.sync_copy(tgt_hbm.at[pl.ds(wid * nstg, nstg)], tgt_v)
        pltpu.sync_copy(lse_hbm, lse_v)
        acc_v = jnp.zeros((_LANES,), jnp.float32)
        cnt_v = jnp.zeros((_LANES,), jnp.float32)

        def gather(g, buf, sem):
            ilist = idx_v.at[pl.ds(g * _LPAD, L)]
            return pltpu.make_async_copy(
                w_hbm.at[ilist], buf.at[pl.ds(0, L)], sem)

        def store_one(g, buf, c, sem):
            b = wid * rows + g
            return pltpu.make_async_copy(
                buf.at[:, c], out_hbm.at[b, :, pl.ds(c * 128, 128)], sem)

        def store_all(g, buf, sem):
            for c in range(8):
                store_one(g, buf, c, sem).start()

        def store_wait(g, buf, sem):
            for c in range(8):
                store_one(g, buf, c, sem).wait()

        gather(0, buf0, gsem0).start()
        gather(1, buf1, gsem1).start()

        def step(i, carries):
            acc_c, cnt_c = carries
            for par in range(2):
                g = 2 * i + par
                buf, gsem, ssem = bufs[par], gsems[par], ssems[par]
                gather(g, buf, gsem).wait()
                for j in range(groups):
                    sl = pl.ds(g * _LPAD + j * _LANES, _LANES)
                    lanes = jnp.arange(_LANES, dtype=jnp.int32) + j * _LANES
                    i16 = jnp.clip(idx_v[sl], 0, V - 1)
                    t16 = tgt_v[sl]
                    m = t16 != -1
                    tsafe = jnp.where(m, t16, 0)
                    row16 = jnp.minimum(lanes, L - 1)
                    wit = plsc.load_gather(
                        buf, [row16, tsafe >> 7, tsafe & 127])
                    ls16 = plsc.load_gather(lse_v, [i16])
                    acc_c = acc_c + jnp.where(m, wit - ls16, 0.0)
                    cnt_c = cnt_c + jnp.where(m, 1.0, 0.0)
                store_all(g, buf, ssem)

                @pl.when(g + 2 < rows)
                def _refill():
                    store_wait(g, buf, ssem)
                    gather(g + 2, buf, gsem).start()

            return (acc_c, cnt_c)

        acc_v, cnt_v = lax.fori_loop(0, rows // 2, step, (acc_v, cnt_v))
        store_wait(rows - 2, buf0, ssem0)
        store_wait(rows - 1, buf1, ssem1)
        for j in range(1024 // _LANES):
            z = jnp.zeros((_LANES,), jnp.float32)
            acc_s[pl.ds(j * _LANES, _LANES)] = z
            cnt_s[pl.ds(j * _LANES, _LANES)] = z
        acc_s[pl.ds(0, _LANES)] = acc_v
        cnt_s[pl.ds(0, _LANES)] = cnt_v
        pltpu.sync_copy(acc_s, acc_hbm.at[pl.ds(wid * 1024, 1024)])
        pltpu.sync_copy(cnt_s, cnt_hbm.at[pl.ds(wid * 1024, 1024)])

    return k(idx2, tgt2, Wp, lse)


def _fin_body(acc_ref, cnt_ref, out_ref):
    s = jnp.sum(acc_ref[...])
    c = jnp.sum(cnt_ref[...])
    out_ref[...] = jnp.full((1, 1), -(s / jnp.maximum(c, 1.0)), jnp.float32)


def _finalize(acc, cnt):
    return pl.pallas_call(
        _fin_body,
        out_shape=jax.ShapeDtypeStruct((1, 1), jnp.float32),
    )(acc, cnt)


def kernel(idx, targets, W):
    B, L = idx.shape
    V = W.shape[0]
    info = plsc.get_sparse_core_info()
    nw = info.num_cores * info.num_subcores
    assert B % (2 * nw) == 0

    rows = B // nw
    idx2 = jnp.pad(idx.astype(jnp.int32), ((0, 0), (0, _LPAD - L))
                   ).reshape(nw * rows * _LPAD)
    tgt2 = jnp.pad(targets.astype(jnp.int32), ((0, 0), (0, _LPAD - L)),
                   constant_values=-1).reshape(nw * rows * _LPAD)
    Wp = jnp.pad(W, ((0, 0), (0, _VPAD - V))).reshape(V, 8, 128)
    lse = jnp.pad(_row_lse(W), (0, _VPAD - V))
    raw, acc, cnt = _sc_gather(idx2, tgt2, Wp, lse, B=B, L=L, nw=nw)
    logits = raw[:, :L, :V]
    loss = _finalize(acc, cnt)[0, 0]
    return logits, loss
